# baseline (speedup 1.0000x reference)
import jax
import jax.numpy as jnp
from jax.experimental import pallas as pl
from jax.experimental.pallas import tpu as pltpu

HIDDEN = 2048
EXPERTS = 16
BT = 1024
NBUF = 5
LOOKAHEAD = 3


def _compute(xb, wt):
    logits = jnp.dot(xb, wt, preferred_element_type=jnp.float32)
    lt = logits.T
    m = jnp.max(lt, axis=0, keepdims=True)
    e = jnp.exp(lt - m)
    s = jnp.sum(e, axis=0, keepdims=True)
    pt = e * (1.0 / s)

    plsum = jnp.sum(pt * lt, axis=0, keepdims=True)
    ent = m + jnp.log(s) - plsum

    rows = jax.lax.broadcasted_iota(jnp.int32, (EXPERTS, BT), 0).astype(jnp.float32)
    w1 = jnp.max(pt, axis=0, keepdims=True)
    i1 = jnp.min(jnp.where(pt == w1, rows, float(EXPERTS)), axis=0, keepdims=True)
    masked = jnp.where(rows == i1, -jnp.inf, pt)
    w2 = jnp.max(masked, axis=0, keepdims=True)
    i2 = jnp.min(jnp.where(masked == w2, rows, float(EXPERTS)), axis=0, keepdims=True)

    rt = 1.0 / (w1 + w2 + 1e-9)
    zero = jnp.zeros((3, BT), jnp.float32)
    strip = jnp.concatenate([w1 * rt, w2 * rt, i1, i2, ent, zero], axis=0)
    return logits, pt.T, strip.T


def _router_block(x_hbm, wt_ref, logits_hbm, probs_hbm, pack_hbm,
                  xbuf, lbuf, pbuf, kbuf, in_sems, out_sems):
    i = pl.program_id(0)
    nblk = pl.num_programs(0)
    slot = jax.lax.rem(i, NBUF)

    def in_copy(blk):
        s = jax.lax.rem(blk, NBUF)
        return pltpu.make_async_copy(
            x_hbm.at[pl.ds(blk * BT, BT), :], xbuf.at[s], in_sems.at[s])

    def out_copies(blk):
        s = jax.lax.rem(blk, NBUF)
        rows = pl.ds(blk * BT, BT)
        return (
            pltpu.make_async_copy(lbuf.at[s], logits_hbm.at[rows, :], out_sems.at[0, s]),
            pltpu.make_async_copy(pbuf.at[s], probs_hbm.at[rows, :], out_sems.at[1, s]),
            pltpu.make_async_copy(kbuf.at[s], pack_hbm.at[rows, :], out_sems.at[2, s]),
        )

    @pl.when(i == 0)
    def _():
        for k in range(LOOKAHEAD):
            in_copy(k).start()

    @pl.when(i + LOOKAHEAD < nblk)
    def _():
        in_copy(i + LOOKAHEAD).start()

    @pl.when(i >= NBUF)
    def _():
        for c in out_copies(i - NBUF):
            c.wait()

    in_copy(i).wait()
    lbuf[slot], pbuf[slot], kbuf[slot] = _compute(xbuf[slot], wt_ref[...])
    for c in out_copies(i):
        c.start()

    @pl.when(i == nblk - 1)
    def _():
        for back in range(NBUF):
            blk = i - back

            @pl.when(blk >= jnp.maximum(nblk - NBUF, 0))
            def _(blk=blk):
                for c in out_copies(blk):
                    c.wait()


def kernel(x, W):
    b, s, h = x.shape
    T = b * s
    x_flat = x.reshape(T, h)
    wt = W.T

    grid = (T // BT,)
    out_shapes = (
        jax.ShapeDtypeStruct((T, EXPERTS), jnp.float32),
        jax.ShapeDtypeStruct((T, EXPERTS), jnp.float32),
        jax.ShapeDtypeStruct((T, 8), jnp.float32),
    )
    hbm = pl.BlockSpec(memory_space=pltpu.MemorySpace.HBM)
    logits, probs, pack = pl.pallas_call(
        _router_block,
        grid=grid,
        in_specs=[
            hbm,
            pl.BlockSpec((HIDDEN, EXPERTS), lambda i: (0, 0)),
        ],
        out_specs=(hbm, hbm, hbm),
        out_shape=out_shapes,
        scratch_shapes=[
            pltpu.MemorySpace.VMEM((NBUF, BT, HIDDEN), jnp.float32),
            pltpu.MemorySpace.VMEM((NBUF, BT, EXPERTS), jnp.float32),
            pltpu.MemorySpace.VMEM((NBUF, BT, EXPERTS), jnp.float32),
            pltpu.MemorySpace.VMEM((NBUF, BT, 8), jnp.float32),
            pltpu.SemaphoreType.DMA((NBUF,)),
            pltpu.SemaphoreType.DMA((3, NBUF)),
        ],
        compiler_params=pltpu.CompilerParams(
            dimension_semantics=("arbitrary",),
        ),
    )(x_flat, wt)

    tw = pack[:, 0:2]
    ti = pack[:, 2:4].astype(jnp.int32)
    entropy = pack[:, 4]
    return (tw, ti, probs, probs, logits, entropy)


# speedup vs baseline: 1.0013x; 1.0013x over previous
import jax
import jax.numpy as jnp
from jax.experimental import pallas as pl
from jax.experimental.pallas import tpu as pltpu

HIDDEN = 2048
EXPERTS = 16
BT = 2048


def _router_block(x_ref, wt_ref, logits_ref, probs_ref, pack_ref):
    logits = jnp.dot(x_ref[...], wt_ref[...], preferred_element_type=jnp.float32)
    logits_ref[...] = logits

    lt = logits.T
    m = jnp.max(lt, axis=0, keepdims=True)
    e = jnp.exp(lt - m)
    s = jnp.sum(e, axis=0, keepdims=True)
    pt = e * (1.0 / s)
    probs_ref[...] = pt.T

    plsum = jnp.sum(pt * lt, axis=0, keepdims=True)
    ent = m + jnp.log(s) - plsum

    rows = jax.lax.broadcasted_iota(jnp.int32, (EXPERTS, BT), 0).astype(jnp.float32)
    w1 = jnp.max(pt, axis=0, keepdims=True)
    i1 = jnp.min(jnp.where(pt == w1, rows, float(EXPERTS)), axis=0, keepdims=True)
    masked = jnp.where(rows == i1, -jnp.inf, pt)
    w2 = jnp.max(masked, axis=0, keepdims=True)
    i2 = jnp.min(jnp.where(masked == w2, rows, float(EXPERTS)), axis=0, keepdims=True)

    rt = 1.0 / (w1 + w2 + 1e-9)
    zero = jnp.zeros((3, BT), jnp.float32)
    strip = jnp.concatenate([w1 * rt, w2 * rt, i1, i2, ent, zero], axis=0)
    pack_ref[...] = strip.T


def kernel(x, W):
    b, s, h = x.shape
    T = b * s
    x_flat = x.reshape(T, h)
    wt = W.T

    grid = (T // BT,)
    out_shapes = (
        jax.ShapeDtypeStruct((T, EXPERTS), jnp.float32),
        jax.ShapeDtypeStruct((T, EXPERTS), jnp.float32),
        jax.ShapeDtypeStruct((T, 8), jnp.float32),
    )
    tok_spec = lambda w: pl.BlockSpec((BT, w), lambda i: (i, 0))
    logits, probs, pack = pl.pallas_call(
        _router_block,
        grid=grid,
        in_specs=[
            tok_spec(HIDDEN),
            pl.BlockSpec((HIDDEN, EXPERTS), lambda i: (0, 0)),
        ],
        out_specs=(
            tok_spec(EXPERTS),
            tok_spec(EXPERTS),
            tok_spec(8),
        ),
        out_shape=out_shapes,
        compiler_params=pltpu.CompilerParams(
            dimension_semantics=("parallel",),
        ),
    )(x_flat, wt)

    tw = pack[:, 0:2]
    ti = pack[:, 2:4].astype(jnp.int32)
    entropy = pack[:, 4]
    return (tw, ti, probs, probs, logits, entropy)


# speedup vs baseline: 1.0261x; 1.0248x over previous
import jax
import jax.numpy as jnp
from jax.experimental import pallas as pl
from jax.experimental.pallas import tpu as pltpu

HIDDEN = 2048
EXPERTS = 16
BT = 256
NBUF = 12
LOOKAHEAD = 10


def _compute(xb, wt):
    logits = jnp.dot(xb, wt, preferred_element_type=jnp.float32)
    lt = logits.T
    m = jnp.max(lt, axis=0, keepdims=True)
    e = jnp.exp(lt - m)
    s = jnp.sum(e, axis=0, keepdims=True)
    pt = e * (1.0 / s)

    plsum = jnp.sum(pt * lt, axis=0, keepdims=True)
    ent = m + jnp.log(s) - plsum

    rows = jax.lax.broadcasted_iota(jnp.int32, (EXPERTS, BT), 0).astype(jnp.float32)
    w1 = jnp.max(pt, axis=0, keepdims=True)
    i1 = jnp.min(jnp.where(pt == w1, rows, float(EXPERTS)), axis=0, keepdims=True)
    masked = jnp.where(rows == i1, -jnp.inf, pt)
    w2 = jnp.max(masked, axis=0, keepdims=True)
    i2 = jnp.min(jnp.where(masked == w2, rows, float(EXPERTS)), axis=0, keepdims=True)

    rt = 1.0 / (w1 + w2 + 1e-9)
    zero = jnp.zeros((3, BT), jnp.float32)
    strip = jnp.concatenate([w1 * rt, w2 * rt, i1, i2, ent, zero], axis=0)
    return logits, pt.T, strip.T


def _router_block(xa_ref, wt_ref, x_hbm,
                  la_ref, pa_ref, ka_ref, lb_hbm, pb_hbm, kb_hbm,
                  xbuf, lbuf, pbuf, kbuf, in_sems, out_sems):
    i = pl.program_id(0)
    nblk = pl.num_programs(0)
    slot = jax.lax.rem(i, NBUF)
    half = nblk * BT

    def in_copy(blk):
        s = jax.lax.rem(blk, NBUF)
        return pltpu.make_async_copy(
            x_hbm.at[pl.ds(half + blk * BT, BT), :], xbuf.at[s], in_sems.at[s])

    def out_copies(blk):
        s = jax.lax.rem(blk, NBUF)
        rows = pl.ds(blk * BT, BT)
        return (
            pltpu.make_async_copy(lbuf.at[s], lb_hbm.at[rows, :], out_sems.at[0, s]),
            pltpu.make_async_copy(pbuf.at[s], pb_hbm.at[rows, :], out_sems.at[1, s]),
            pltpu.make_async_copy(kbuf.at[s], kb_hbm.at[rows, :], out_sems.at[2, s]),
        )

    @pl.when(i == 0)
    def _():
        for k in range(LOOKAHEAD):
            in_copy(k).start()

    @pl.when(i + LOOKAHEAD < nblk)
    def _():
        in_copy(i + LOOKAHEAD).start()

    @pl.when(i >= NBUF)
    def _():
        for c in out_copies(i - NBUF):
            c.wait()

    wt = wt_ref[...]

    la_ref[...], pa_ref[...], ka_ref[...] = _compute(xa_ref[...], wt)

    in_copy(i).wait()
    lbuf[slot], pbuf[slot], kbuf[slot] = _compute(xbuf[slot], wt)
    for c in out_copies(i):
        c.start()

    @pl.when(i == nblk - 1)
    def _():
        for back in range(NBUF):
            blk = i - back

            @pl.when(blk >= jnp.maximum(nblk - NBUF, 0))
            def _(blk=blk):
                for c in out_copies(blk):
                    c.wait()


def kernel(x, W):
    b, s, h = x.shape
    T = b * s
    half = T // 2
    x_flat = x.reshape(T, h)
    wt = W.T

    grid = (half // BT,)
    out_shapes = (
        jax.ShapeDtypeStruct((half, EXPERTS), jnp.float32),
        jax.ShapeDtypeStruct((half, EXPERTS), jnp.float32),
        jax.ShapeDtypeStruct((half, 8), jnp.float32),
        jax.ShapeDtypeStruct((half, EXPERTS), jnp.float32),
        jax.ShapeDtypeStruct((half, EXPERTS), jnp.float32),
        jax.ShapeDtypeStruct((half, 8), jnp.float32),
    )
    hbm = pl.BlockSpec(memory_space=pltpu.MemorySpace.HBM)
    tok_spec = lambda w: pl.BlockSpec((BT, w), lambda i: (i, 0))
    la, pa, ka, lb, pb, kb = pl.pallas_call(
        _router_block,
        grid=grid,
        in_specs=[
            tok_spec(HIDDEN),
            pl.BlockSpec((HIDDEN, EXPERTS), lambda i: (0, 0)),
            hbm,
        ],
        out_specs=(
            tok_spec(EXPERTS), tok_spec(EXPERTS), tok_spec(8),
            hbm, hbm, hbm,
        ),
        out_shape=out_shapes,
        scratch_shapes=[
            pltpu.MemorySpace.VMEM((NBUF, BT, HIDDEN), jnp.float32),
            pltpu.MemorySpace.VMEM((NBUF, BT, EXPERTS), jnp.float32),
            pltpu.MemorySpace.VMEM((NBUF, BT, EXPERTS), jnp.float32),
            pltpu.MemorySpace.VMEM((NBUF, BT, 8), jnp.float32),
            pltpu.SemaphoreType.DMA((NBUF,)),
            pltpu.SemaphoreType.DMA((3, NBUF)),
        ],
        compiler_params=pltpu.CompilerParams(
            dimension_semantics=("arbitrary",),
        ),
    )(x_flat, wt, x_flat)

    logits = jnp.concatenate([la, lb], axis=0)
    probs = jnp.concatenate([pa, pb], axis=0)
    pack = jnp.concatenate([ka, kb], axis=0)
    tw = pack[:, 0:2]
    ti = pack[:, 2:4].astype(jnp.int32)
    entropy = pack[:, 4]
    return (tw, ti, probs, probs, logits, entropy)
